# Initial kernel scaffold; baseline (speedup 1.0000x reference)
import jax
import jax.numpy as jnp
from jax.experimental import pallas as pl


def kernel(x, emb, W, b):
    raise NotImplementedError("write your pallas kernel here")


# speedup vs baseline: 12.8947x; 12.8947x over previous
import functools

import jax
import jax.numpy as jnp
from jax import lax
from jax.experimental import pallas as pl
from jax.experimental.pallas import tpu as pltpu
from jax.experimental.pallas import tpu_sc as plsc


def _seq_chunks(seq):
    chunks = []
    off = 0
    while off < seq:
        size = min(128, seq - off)
        chunks.append((off, size))
        off += size
    return chunks


def _make_sc_pool(batch, seq, hid, n_words):
    mesh = plsc.VectorSubcoreMesh(core_axis_name="c", subcore_axis_name="s")
    nw = mesh.num_cores * mesh.num_subcores
    assert batch % nw == 0
    bpw = batch // nw
    assert seq % 8 == 0 and hid % 16 == 0
    chunks = _seq_chunks(seq)
    n_vec = hid // 16
    assert seq % 8 == 0
    n_outer = seq // 8

    @functools.partial(
        pl.kernel,
        mesh=mesh,
        out_type=jax.ShapeDtypeStruct((batch, hid), jnp.float32),
        scratch_types=[
            pltpu.VMEM((bpw, seq), jnp.int32),
            pltpu.VMEM((2, seq, hid), jnp.float32),
            pltpu.VMEM((bpw, hid), jnp.float32),
            pltpu.SemaphoreType.DMA,
            pltpu.SemaphoreType.DMA,
        ],
        compiler_params=pltpu.CompilerParams(use_tc_tiling_on_sc=False),
    )
    def sc_pool(xt_hbm, emb_hbm, out_hbm, idx_v, rows_v, acc_v, sem0, sem1):
        ncores = mesh.num_cores
        wid = lax.axis_index("s") * ncores + lax.axis_index("c")
        base = wid * bpw

        pltpu.sync_copy(xt_hbm.at[pl.ds(base, bpw)], idx_v)

        sems = (sem0, sem1)

        def fire(b, buf):
            for off, size in chunks:
                pltpu.make_async_copy(
                    emb_hbm.at[idx_v.at[b, pl.ds(off, size)]],
                    rows_v.at[buf, pl.ds(off, size)],
                    sems[buf],
                ).start()

        def drain(buf):
            for off, size in chunks:
                pltpu.make_async_copy(
                    emb_hbm.at[idx_v.at[0, pl.ds(off, size)]],
                    rows_v.at[buf, pl.ds(off, size)],
                    sems[buf],
                ).wait()

        def reduce(b, buf):
            zero = jnp.zeros((16,), jnp.float32)

            def body(so, carry):
                acc = list(carry)
                s0 = so * 8
                for r in range(8):
                    for k in range(n_vec):
                        acc[k] = acc[k] + rows_v[buf, s0 + r, pl.ds(k * 16, 16)]
                return tuple(acc)

            acc = lax.fori_loop(0, n_outer, body, (zero,) * n_vec)
            for k in range(n_vec):
                acc_v[b, pl.ds(k * 16, 16)] = acc[k]

        fire(0, 0)

        @pl.loop(0, bpw, step=2)
        def _(i):
            fire(i + 1, 1)
            drain(0)
            reduce(i, 0)

            @pl.when(i + 2 < bpw)
            def _():
                fire(i + 2, 0)

            drain(1)
            reduce(i + 1, 1)

        pltpu.sync_copy(acc_v, out_hbm.at[pl.ds(base, bpw)])

    return sc_pool


def _mm_body(inv_seq, h_ref, w_ref, b_ref, o_ref):
    h = h_ref[...] * inv_seq
    o_ref[...] = (
        jnp.dot(h, w_ref[...], preferred_element_type=jnp.float32) + b_ref[...]
    )


def _matmul(hsum, w, b2, inv_seq):
    batch, hid = hsum.shape
    n_class = w.shape[1]
    bm = 512
    grid = (batch // bm,)
    return pl.pallas_call(
        functools.partial(_mm_body, inv_seq),
        grid=grid,
        in_specs=[
            pl.BlockSpec((bm, hid), lambda i: (i, 0)),
            pl.BlockSpec((hid, n_class), lambda i: (0, 0)),
            pl.BlockSpec((1, n_class), lambda i: (0, 0)),
        ],
        out_specs=pl.BlockSpec((bm, n_class), lambda i: (i, 0)),
        out_shape=jax.ShapeDtypeStruct((batch, n_class), jnp.float32),
    )(hsum, w, b2)


def kernel(x, emb, W, b):
    seq, batch = x.shape
    n_words, hid = emb.shape
    xt = jnp.transpose(x).astype(jnp.int32)
    sc_pool = _make_sc_pool(batch, seq, hid, n_words)
    hsum = sc_pool(xt, emb)
    y = _matmul(hsum, W, b.reshape(1, -1), 1.0 / seq)
    return y
